# baseline (speedup 1.0000x reference)
import functools

import jax
import jax.numpy as jnp
from jax import lax
from jax.experimental import pallas as pl
from jax.experimental.pallas import tpu as pltpu
from jax.experimental.pallas import tpu_sc as plsc

_NC = 2
_NS = 16
_LANES = 16


def _sc_lookup_native(n_tiles, tpr, v, d, tpb):
    nw = _NC * _NS
    per_w = n_tiles // nw
    n_blk = per_w // tpb
    tile_sz = d * 128
    mesh = plsc.VectorSubcoreMesh(core_axis_name="c", subcore_axis_name="s")

    @functools.partial(
        pl.kernel,
        out_type=jax.ShapeDtypeStruct((n_tiles * tile_sz,), jnp.float32),
        mesh=mesh,
        scratch_types=[
            pltpu.VMEM((v * d * _LANES,), jnp.float32),
            pltpu.VMEM((tpb, 1, 128), jnp.int32),
            pltpu.VMEM((tpb, 1, 128), jnp.int32),
            pltpu.VMEM((tpb * tile_sz,), jnp.float32),
            pltpu.VMEM((tpb * tile_sz,), jnp.float32),
            pltpu.SemaphoreType.DMA,
            pltpu.SemaphoreType.DMA,
            pltpu.SemaphoreType.DMA,
            pltpu.SemaphoreType.DMA,
            pltpu.SemaphoreType.DMA,
        ],
        compiler_params=pltpu.CompilerParams(
            needs_layout_passes=False, use_tc_tiling_on_sc=False),
    )
    def k(idx_hbm, emb_hbm, out_hbm, table_v, idx_v0, idx_v1, rows_v0,
          rows_v1, sem_i0, sem_i1, sem_o0, sem_o1, sem_t):
        idxs_v = (idx_v0, idx_v1)
        rows_v = (rows_v0, rows_v1)
        sem_i = (sem_i0, sem_i1)
        sem_o = (sem_o0, sem_o1)
        wid = lax.axis_index("s") * _NC + lax.axis_index("c")
        tile0 = wid * per_w

        tcp = pltpu.make_async_copy(emb_hbm, table_v, sem_t)
        tcp.start()

        def in_cp(b, buf):
            u0 = tile0 + b * tpb
            j = u0 // tpr
            t0 = u0 % tpr
            jj = j // 8
            j8 = j % 8
            return pltpu.make_async_copy(
                idx_hbm.at[pl.ds(jj * tpr + t0, tpb), pl.ds(j8, 1), :],
                idxs_v[buf], sem_i[buf])

        def out_cp(b, buf):
            return pltpu.make_async_copy(
                rows_v[buf],
                out_hbm.at[pl.ds((tile0 + b * tpb) * tile_sz, tpb * tile_sz)],
                sem_o[buf])

        in_cp(0, 0).start()
        tcp.wait()

        iota = lax.iota(jnp.int32, _LANES)
        kvecs = [kk * _LANES + iota for kk in range(d)]

        def compute(cur):
            @plsc.parallel_loop(0, tpb, 1, unroll=4)
            def _(tt):
                obase = tt * tile_sz
                for li in range(8):
                    iv = idxs_v[cur][tt, 0, pl.ds(li * _LANES, _LANES)]
                    srcb = iv * (d * _LANES)
                    for kk in range(d):
                        vals = plsc.load_gather(table_v, [srcb + kvecs[kk]])
                        rows_v[cur][
                            pl.ds(obase + kk * 128 + li * _LANES, _LANES)
                        ] = vals

        def do_block(b, cur):
            @pl.when(b + 1 < n_blk)
            def _():
                in_cp(b + 1, 1 - cur).start()

            in_cp(b, cur).wait()

            @pl.when(b >= 2)
            def _():
                out_cp(b - 2, cur).wait()

            compute(cur)
            out_cp(b, cur).start()

        def block_pair(i, _):
            for j in range(2):
                do_block(i * 2 + j, j)
            return 0

        lax.fori_loop(0, n_blk // 2, block_pair, 0)
        if n_blk % 2:
            do_block(n_blk - 1, 0)
        out_cp(n_blk - 2, (n_blk - 2) % 2).wait()
        out_cp(n_blk - 1, (n_blk - 1) % 2).wait()

    return k


def _sc_lookup_flat(n_pad, v, d, blk):
    nw = _NC * _NS
    per_w = n_pad // nw
    n_blk = per_w // blk
    groups = blk // _LANES
    mesh = plsc.VectorSubcoreMesh(core_axis_name="c", subcore_axis_name="s")

    @functools.partial(
        pl.kernel,
        out_type=jax.ShapeDtypeStruct((n_pad * d,), jnp.float32),
        mesh=mesh,
        scratch_types=[
            pltpu.VMEM((v * d,), jnp.float32),
            pltpu.VMEM((blk,), jnp.int32),
            pltpu.VMEM((blk,), jnp.int32),
            pltpu.VMEM((blk * d,), jnp.float32),
            pltpu.VMEM((blk * d,), jnp.float32),
            pltpu.SemaphoreType.DMA,
            pltpu.SemaphoreType.DMA,
            pltpu.SemaphoreType.DMA,
            pltpu.SemaphoreType.DMA,
            pltpu.SemaphoreType.DMA,
        ],
        compiler_params=pltpu.CompilerParams(
            needs_layout_passes=False, use_tc_tiling_on_sc=False),
    )
    def k(idx_hbm, emb_hbm, out_hbm, table_v, idx_v0, idx_v1, rows_v0,
          rows_v1, sem_i0, sem_i1, sem_o0, sem_o1, sem_t):
        idxs_v = (idx_v0, idx_v1)
        rows_v = (rows_v0, rows_v1)
        sem_i = (sem_i0, sem_i1)
        sem_o = (sem_o0, sem_o1)
        wid = lax.axis_index("s") * _NC + lax.axis_index("c")
        base = wid * per_w

        tcp = pltpu.make_async_copy(emb_hbm, table_v, sem_t)
        tcp.start()

        def in_cp(b, buf):
            return pltpu.make_async_copy(
                idx_hbm.at[pl.ds(base + b * blk, blk)], idxs_v[buf],
                sem_i[buf])

        def out_cp(b, buf):
            return pltpu.make_async_copy(
                rows_v[buf],
                out_hbm.at[pl.ds((base + b * blk) * d, blk * d)],
                sem_o[buf])

        in_cp(0, 0).start()
        tcp.wait()

        iota = lax.iota(jnp.int32, _LANES)
        pos0 = iota * d

        def compute(cur):
            @plsc.parallel_loop(0, groups, 1, unroll=8)
            def _(g):
                iv = idxs_v[cur][pl.ds(g * _LANES, _LANES)]
                srcb = iv * d
                gbase = g * (_LANES * d)
                for dd in range(d):
                    vals = plsc.load_gather(table_v, [srcb + dd])
                    plsc.store_scatter(
                        rows_v[cur], [pos0 + (gbase + dd)], vals)

        def block_pair(i, _):
            for j in range(2):
                b = i * 2 + j
                cur = j

                @pl.when(b + 1 < n_blk)
                def _():
                    in_cp(b + 1, 1 - j).start()

                in_cp(b, cur).wait()

                @pl.when(b >= 2)
                def _():
                    out_cp(b - 2, cur).wait()

                compute(cur)
                out_cp(b, cur).start()
            return 0

        lax.fori_loop(0, n_blk // 2, block_pair, 0)
        out_cp(n_blk - 2, (n_blk - 2) % 2).wait()
        out_cp(n_blk - 1, (n_blk - 1) % 2).wait()

    return k


def kernel(idx, emb):
    r, c = idx.shape
    v, d = emb.shape
    n = r * c
    nw = _NC * _NS

    if d == 8 and r % 128 == 0 and c % 8 == 0 and idx.dtype == jnp.int32:
        tpr = r // 128
        n_tiles = c * tpr
        if n_tiles % nw == 0:
            per_w = n_tiles // nw
            tpb = None
            for cand in (128, 64, 32, 16, 8, 4, 2, 1):
                if tpr % cand == 0 and per_w % cand == 0 \
                        and per_w // cand >= 2 \
                        and cand * (128 + d * 128) * 4 * 2 <= 460 * 1024:
                    tpb = cand
                    break
            if tpb is not None:
                idx_n = (idx.transpose(1, 0)
                         .reshape(c // 8, 8, tpr, 128)
                         .transpose(0, 2, 1, 3)
                         .reshape((c // 8) * tpr, 8, 128))
                emb16 = jnp.broadcast_to(
                    emb.reshape(v * d, 1).astype(jnp.float32),
                    (v * d, _LANES)).reshape(v * d * _LANES)
                q = _sc_lookup_native(n_tiles, tpr, v, d, tpb)(idx_n, emb16)
                return (q.reshape(c, tpr, d, 128)
                        .transpose(1, 3, 0, 2)
                        .reshape(r, c, d))

    blk = 16
    for cand in (4096, 2048, 1024, 512, 256, 128, 64, 32, 16):
        per_w = -(-n // nw)
        if n % nw == 0 and per_w % cand == 0 and (per_w // cand) % 2 == 0:
            blk = cand
            break
    chunk = nw * blk * 2
    n_pad = ((n + chunk - 1) // chunk) * chunk

    idx_flat = idx.reshape(n).astype(jnp.int32)
    if n_pad != n:
        idx_flat = jnp.pad(idx_flat, (0, n_pad - n))
    emb_flat = emb.reshape(v * d).astype(jnp.float32)

    out = _sc_lookup_flat(n_pad, v, d, blk)(idx_flat, emb_flat)
    return out[: n * d].reshape(r, c, d)


# speedup vs baseline: 1.1972x; 1.1972x over previous
import functools

import jax
import jax.numpy as jnp
from jax import lax
from jax.experimental import pallas as pl
from jax.experimental.pallas import tpu as pltpu
from jax.experimental.pallas import tpu_sc as plsc

_NC = 2
_NS = 16
_LANES = 16


def _sc_lookup_native(n_tiles, tpr, v, d, tpb):
    nw = _NC * _NS
    per_w = n_tiles // nw
    n_blk = per_w // tpb
    tile_sz = d * 128
    mesh = plsc.VectorSubcoreMesh(core_axis_name="c", subcore_axis_name="s")

    @functools.partial(
        pl.kernel,
        out_type=jax.ShapeDtypeStruct((n_tiles * tile_sz,), jnp.float32),
        mesh=mesh,
        scratch_types=[
            pltpu.VMEM((v * d * _LANES,), jnp.float32),
            pltpu.VMEM((tpb, 1, 128), jnp.int32),
            pltpu.VMEM((tpb, 1, 128), jnp.int32),
            pltpu.VMEM((tpb * tile_sz,), jnp.float32),
            pltpu.VMEM((tpb * tile_sz,), jnp.float32),
            pltpu.SemaphoreType.DMA,
            pltpu.SemaphoreType.DMA,
            pltpu.SemaphoreType.DMA,
            pltpu.SemaphoreType.DMA,
            pltpu.SemaphoreType.DMA,
        ],
        compiler_params=pltpu.CompilerParams(
            needs_layout_passes=False, use_tc_tiling_on_sc=False),
    )
    def k(idx_hbm, emb_hbm, out_hbm, table_v, idx_v0, idx_v1, rows_v0,
          rows_v1, sem_i0, sem_i1, sem_o0, sem_o1, sem_t):
        idxs_v = (idx_v0, idx_v1)
        rows_v = (rows_v0, rows_v1)
        sem_i = (sem_i0, sem_i1)
        sem_o = (sem_o0, sem_o1)
        wid = lax.axis_index("s") * _NC + lax.axis_index("c")
        tile0 = wid * per_w

        tcp = pltpu.make_async_copy(emb_hbm, table_v, sem_t)
        tcp.start()

        def in_cp(b, buf):
            u0 = tile0 + b * tpb
            j = u0 // tpr
            t0 = u0 % tpr
            jj = j // 8
            j8 = j % 8
            return pltpu.make_async_copy(
                idx_hbm.at[pl.ds(jj * tpr + t0, tpb), pl.ds(j8, 1), :],
                idxs_v[buf], sem_i[buf])

        def out_cp(b, buf):
            return pltpu.make_async_copy(
                rows_v[buf],
                out_hbm.at[pl.ds((tile0 + b * tpb) * tile_sz, tpb * tile_sz)],
                sem_o[buf])

        in_cp(0, 0).start()
        tcp.wait()

        iota = lax.iota(jnp.int32, _LANES)
        kvecs = [kk * _LANES + iota for kk in range(d)]

        def compute(cur):
            @plsc.parallel_loop(0, tpb, 1, unroll=2)
            def _(tt):
                obase = tt * tile_sz
                for li in range(8):
                    iv = idxs_v[cur][tt, 0, pl.ds(li * _LANES, _LANES)]
                    srcb = iv * (d * _LANES)
                    for kk in range(d):
                        vals = plsc.load_gather(table_v, [srcb + kvecs[kk]])
                        rows_v[cur][
                            pl.ds(obase + kk * 128 + li * _LANES, _LANES)
                        ] = vals

        def do_block(b, cur):
            @pl.when(b + 1 < n_blk)
            def _():
                in_cp(b + 1, 1 - cur).start()

            in_cp(b, cur).wait()

            @pl.when(b >= 2)
            def _():
                out_cp(b - 2, cur).wait()

            compute(cur)
            out_cp(b, cur).start()

        def block_pair(i, _):
            for j in range(2):
                do_block(i * 2 + j, j)
            return 0

        lax.fori_loop(0, n_blk // 2, block_pair, 0)
        if n_blk % 2:
            do_block(n_blk - 1, 0)
        out_cp(n_blk - 2, (n_blk - 2) % 2).wait()
        out_cp(n_blk - 1, (n_blk - 1) % 2).wait()

    return k


def _sc_lookup_slab(cg, tpr, v, d, tb):
    nw = _NC * _NS
    jt_total = cg * tpr
    per_w = jt_total // nw
    n_blk = per_w // tb
    tile_sz = d * 128
    mesh = plsc.VectorSubcoreMesh(core_axis_name="c", subcore_axis_name="s")

    @functools.partial(
        pl.kernel,
        out_type=jax.ShapeDtypeStruct((cg, 8, tpr, tile_sz), jnp.float32),
        mesh=mesh,
        scratch_types=[
            pltpu.VMEM((v * d * _LANES,), jnp.float32),
            pltpu.VMEM((tb, 8, 128), jnp.int32),
            pltpu.VMEM((tb, 8, 128), jnp.int32),
            pltpu.VMEM((8, tb, tile_sz), jnp.float32),
            pltpu.VMEM((8, tb, tile_sz), jnp.float32),
            pltpu.SemaphoreType.DMA,
            pltpu.SemaphoreType.DMA,
            pltpu.SemaphoreType.DMA,
            pltpu.SemaphoreType.DMA,
            pltpu.SemaphoreType.DMA,
        ],
        compiler_params=pltpu.CompilerParams(
            needs_layout_passes=False, use_tc_tiling_on_sc=False),
    )
    def k(idx_hbm, emb_hbm, out_hbm, table_v, idx_v0, idx_v1, rows_v0,
          rows_v1, sem_i0, sem_i1, sem_o0, sem_o1, sem_t):
        idxs_v = (idx_v0, idx_v1)
        rows_v = (rows_v0, rows_v1)
        sem_i = (sem_i0, sem_i1)
        sem_o = (sem_o0, sem_o1)
        wid = lax.axis_index("s") * _NC + lax.axis_index("c")
        jt0 = wid * per_w

        tcp = pltpu.make_async_copy(emb_hbm, table_v, sem_t)
        tcp.start()

        def in_cp(b, buf):
            return pltpu.make_async_copy(
                idx_hbm.at[pl.ds(jt0 + b * tb, tb), :, :],
                idxs_v[buf], sem_i[buf])

        def out_cp(b, buf):
            jt = jt0 + b * tb
            jj = jt // tpr
            t0 = jt % tpr
            return pltpu.make_async_copy(
                rows_v[buf],
                out_hbm.at[jj, :, pl.ds(t0, tb), :],
                sem_o[buf])

        in_cp(0, 0).start()
        tcp.wait()

        iota = lax.iota(jnp.int32, _LANES)
        kvecs = [kk * _LANES + iota for kk in range(d)]

        def compute(cur):
            @plsc.parallel_loop(0, tb * 8, 1, unroll=2)
            def _(q):
                t_off = q // 8
                j8 = q % 8
                for li in range(8):
                    iv = idxs_v[cur][t_off, j8, pl.ds(li * _LANES, _LANES)]
                    srcb = iv * (d * _LANES)
                    for kk in range(d):
                        vals = plsc.load_gather(table_v, [srcb + kvecs[kk]])
                        rows_v[cur][
                            j8, t_off,
                            pl.ds(kk * 128 + li * _LANES, _LANES)
                        ] = vals

        def do_block(b, cur):
            @pl.when(b + 1 < n_blk)
            def _():
                in_cp(b + 1, 1 - cur).start()

            in_cp(b, cur).wait()

            @pl.when(b >= 2)
            def _():
                out_cp(b - 2, cur).wait()

            compute(cur)
            out_cp(b, cur).start()

        def block_pair(i, _):
            for j in range(2):
                do_block(i * 2 + j, j)
            return 0

        lax.fori_loop(0, n_blk // 2, block_pair, 0)
        if n_blk % 2:
            do_block(n_blk - 1, 0)
        out_cp(n_blk - 2, (n_blk - 2) % 2).wait()
        out_cp(n_blk - 1, (n_blk - 1) % 2).wait()

    return k


def _sc_lookup_flat(n_pad, v, d, blk):
    nw = _NC * _NS
    per_w = n_pad // nw
    n_blk = per_w // blk
    groups = blk // _LANES
    mesh = plsc.VectorSubcoreMesh(core_axis_name="c", subcore_axis_name="s")

    @functools.partial(
        pl.kernel,
        out_type=jax.ShapeDtypeStruct((n_pad * d,), jnp.float32),
        mesh=mesh,
        scratch_types=[
            pltpu.VMEM((v * d,), jnp.float32),
            pltpu.VMEM((blk,), jnp.int32),
            pltpu.VMEM((blk,), jnp.int32),
            pltpu.VMEM((blk * d,), jnp.float32),
            pltpu.VMEM((blk * d,), jnp.float32),
            pltpu.SemaphoreType.DMA,
            pltpu.SemaphoreType.DMA,
            pltpu.SemaphoreType.DMA,
            pltpu.SemaphoreType.DMA,
            pltpu.SemaphoreType.DMA,
        ],
        compiler_params=pltpu.CompilerParams(
            needs_layout_passes=False, use_tc_tiling_on_sc=False),
    )
    def k(idx_hbm, emb_hbm, out_hbm, table_v, idx_v0, idx_v1, rows_v0,
          rows_v1, sem_i0, sem_i1, sem_o0, sem_o1, sem_t):
        idxs_v = (idx_v0, idx_v1)
        rows_v = (rows_v0, rows_v1)
        sem_i = (sem_i0, sem_i1)
        sem_o = (sem_o0, sem_o1)
        wid = lax.axis_index("s") * _NC + lax.axis_index("c")
        base = wid * per_w

        tcp = pltpu.make_async_copy(emb_hbm, table_v, sem_t)
        tcp.start()

        def in_cp(b, buf):
            return pltpu.make_async_copy(
                idx_hbm.at[pl.ds(base + b * blk, blk)], idxs_v[buf],
                sem_i[buf])

        def out_cp(b, buf):
            return pltpu.make_async_copy(
                rows_v[buf],
                out_hbm.at[pl.ds((base + b * blk) * d, blk * d)],
                sem_o[buf])

        in_cp(0, 0).start()
        tcp.wait()

        iota = lax.iota(jnp.int32, _LANES)
        pos0 = iota * d

        def compute(cur):
            @plsc.parallel_loop(0, groups, 1, unroll=8)
            def _(g):
                iv = idxs_v[cur][pl.ds(g * _LANES, _LANES)]
                srcb = iv * d
                gbase = g * (_LANES * d)
                for dd in range(d):
                    vals = plsc.load_gather(table_v, [srcb + dd])
                    plsc.store_scatter(
                        rows_v[cur], [pos0 + (gbase + dd)], vals)

        def block_pair(i, _):
            for j in range(2):
                b = i * 2 + j
                cur = j

                @pl.when(b + 1 < n_blk)
                def _():
                    in_cp(b + 1, 1 - j).start()

                in_cp(b, cur).wait()

                @pl.when(b >= 2)
                def _():
                    out_cp(b - 2, cur).wait()

                compute(cur)
                out_cp(b, cur).start()
            return 0

        lax.fori_loop(0, n_blk // 2, block_pair, 0)
        out_cp(n_blk - 2, (n_blk - 2) % 2).wait()
        out_cp(n_blk - 1, (n_blk - 1) % 2).wait()

    return k


def kernel(idx, emb):
    r, c = idx.shape
    v, d = emb.shape
    n = r * c
    nw = _NC * _NS

    if d == 8 and r % 128 == 0 and c % 8 == 0 and idx.dtype == jnp.int32:
        tpr = r // 128
        n_tiles = c * tpr
        cg = c // 8
        jt_total = cg * tpr
        if jt_total % nw == 0:
            per_w_jt = jt_total // nw
            tb = None
            for cand in (4, 2, 1):
                if tpr % cand == 0 and per_w_jt % cand == 0 \
                        and per_w_jt // cand >= 2:
                    tb = cand
                    break
            if tb is not None:
                idx_n = (idx.transpose(1, 0)
                         .reshape(cg, 8, tpr, 128)
                         .transpose(0, 2, 1, 3)
                         .reshape(jt_total, 8, 128))
                emb16 = jnp.broadcast_to(
                    emb.reshape(v * d, 1).astype(jnp.float32),
                    (v * d, _LANES)).reshape(v * d * _LANES)
                q = _sc_lookup_slab(cg, tpr, v, d, tb)(idx_n, emb16)
                return (q.reshape(c, tpr, d, 128)
                        .transpose(1, 3, 0, 2)
                        .reshape(r, c, d))
        if n_tiles % nw == 0:
            per_w = n_tiles // nw
            tpb = None
            for cand in (128, 64, 32, 16, 8, 4, 2, 1):
                if tpr % cand == 0 and per_w % cand == 0 \
                        and per_w // cand >= 2 \
                        and cand * (128 + d * 128) * 4 * 2 <= 460 * 1024:
                    tpb = cand
                    break
            if tpb is not None:
                idx_n = (idx.transpose(1, 0)
                         .reshape(c // 8, 8, tpr, 128)
                         .transpose(0, 2, 1, 3)
                         .reshape((c // 8) * tpr, 8, 128))
                emb16 = jnp.broadcast_to(
                    emb.reshape(v * d, 1).astype(jnp.float32),
                    (v * d, _LANES)).reshape(v * d * _LANES)
                q = _sc_lookup_native(n_tiles, tpr, v, d, tpb)(idx_n, emb16)
                return (q.reshape(c, tpr, d, 128)
                        .transpose(1, 3, 0, 2)
                        .reshape(r, c, d))

    blk = 16
    for cand in (4096, 2048, 1024, 512, 256, 128, 64, 32, 16):
        per_w = -(-n // nw)
        if n % nw == 0 and per_w % cand == 0 and (per_w // cand) % 2 == 0:
            blk = cand
            break
    chunk = nw * blk * 2
    n_pad = ((n + chunk - 1) // chunk) * chunk

    idx_flat = idx.reshape(n).astype(jnp.int32)
    if n_pad != n:
        idx_flat = jnp.pad(idx_flat, (0, n_pad - n))
    emb_flat = emb.reshape(v * d).astype(jnp.float32)

    out = _sc_lookup_flat(n_pad, v, d, blk)(idx_flat, emb_flat)
    return out[: n * d].reshape(r, c, d)


# speedup vs baseline: 1.2053x; 1.0068x over previous
import functools

import jax
import jax.numpy as jnp
from jax import lax
from jax.experimental import pallas as pl
from jax.experimental.pallas import tpu as pltpu
from jax.experimental.pallas import tpu_sc as plsc

_NC = 2
_NS = 16
_LANES = 16


def _sc_lookup_native(n_tiles, tpr, v, d, tpb):
    nw = _NC * _NS
    per_w = n_tiles // nw
    n_blk = per_w // tpb
    tile_sz = d * 128
    mesh = plsc.VectorSubcoreMesh(core_axis_name="c", subcore_axis_name="s")

    @functools.partial(
        pl.kernel,
        out_type=jax.ShapeDtypeStruct((n_tiles * tile_sz,), jnp.float32),
        mesh=mesh,
        scratch_types=[
            pltpu.VMEM((v * d * _LANES,), jnp.float32),
            pltpu.VMEM((tpb, 1, 128), jnp.int32),
            pltpu.VMEM((tpb, 1, 128), jnp.int32),
            pltpu.VMEM((tpb * tile_sz,), jnp.float32),
            pltpu.VMEM((tpb * tile_sz,), jnp.float32),
            pltpu.SemaphoreType.DMA,
            pltpu.SemaphoreType.DMA,
            pltpu.SemaphoreType.DMA,
            pltpu.SemaphoreType.DMA,
            pltpu.SemaphoreType.DMA,
        ],
        compiler_params=pltpu.CompilerParams(
            needs_layout_passes=False, use_tc_tiling_on_sc=False),
    )
    def k(idx_hbm, emb_hbm, out_hbm, table_v, idx_v0, idx_v1, rows_v0,
          rows_v1, sem_i0, sem_i1, sem_o0, sem_o1, sem_t):
        idxs_v = (idx_v0, idx_v1)
        rows_v = (rows_v0, rows_v1)
        sem_i = (sem_i0, sem_i1)
        sem_o = (sem_o0, sem_o1)
        wid = lax.axis_index("s") * _NC + lax.axis_index("c")
        tile0 = wid * per_w

        tcp = pltpu.make_async_copy(emb_hbm, table_v, sem_t)
        tcp.start()

        def in_cp(b, buf):
            u0 = tile0 + b * tpb
            j = u0 // tpr
            t0 = u0 % tpr
            jj = j // 8
            j8 = j % 8
            return pltpu.make_async_copy(
                idx_hbm.at[pl.ds(jj * tpr + t0, tpb), pl.ds(j8, 1), :],
                idxs_v[buf], sem_i[buf])

        def out_cp(b, buf):
            return pltpu.make_async_copy(
                rows_v[buf],
                out_hbm.at[pl.ds((tile0 + b * tpb) * tile_sz, tpb * tile_sz)],
                sem_o[buf])

        in_cp(0, 0).start()
        tcp.wait()

        iota = lax.iota(jnp.int32, _LANES)
        kvecs = [kk * _LANES + iota for kk in range(d)]

        def compute(cur):
            @plsc.parallel_loop(0, tpb, 1, unroll=2)
            def _(tt):
                obase = tt * tile_sz
                for li in range(8):
                    iv = idxs_v[cur][tt, 0, pl.ds(li * _LANES, _LANES)]
                    srcb = iv * (d * _LANES)
                    for kk in range(d):
                        vals = plsc.load_gather(table_v, [srcb + kvecs[kk]])
                        rows_v[cur][
                            pl.ds(obase + kk * 128 + li * _LANES, _LANES)
                        ] = vals

        def do_block(b, cur):
            @pl.when(b + 1 < n_blk)
            def _():
                in_cp(b + 1, 1 - cur).start()

            in_cp(b, cur).wait()

            @pl.when(b >= 2)
            def _():
                out_cp(b - 2, cur).wait()

            compute(cur)
            out_cp(b, cur).start()

        def block_pair(i, _):
            for j in range(2):
                do_block(i * 2 + j, j)
            return 0

        lax.fori_loop(0, n_blk // 2, block_pair, 0)
        if n_blk % 2:
            do_block(n_blk - 1, 0)
        out_cp(n_blk - 2, (n_blk - 2) % 2).wait()
        out_cp(n_blk - 1, (n_blk - 1) % 2).wait()

    return k


def _sc_lookup_flat(n_pad, v, d, blk):
    nw = _NC * _NS
    per_w = n_pad // nw
    n_blk = per_w // blk
    groups = blk // _LANES
    mesh = plsc.VectorSubcoreMesh(core_axis_name="c", subcore_axis_name="s")

    @functools.partial(
        pl.kernel,
        out_type=jax.ShapeDtypeStruct((n_pad * d,), jnp.float32),
        mesh=mesh,
        scratch_types=[
            pltpu.VMEM((v * d,), jnp.float32),
            pltpu.VMEM((blk,), jnp.int32),
            pltpu.VMEM((blk,), jnp.int32),
            pltpu.VMEM((blk * d,), jnp.float32),
            pltpu.VMEM((blk * d,), jnp.float32),
            pltpu.SemaphoreType.DMA,
            pltpu.SemaphoreType.DMA,
            pltpu.SemaphoreType.DMA,
            pltpu.SemaphoreType.DMA,
            pltpu.SemaphoreType.DMA,
        ],
        compiler_params=pltpu.CompilerParams(
            needs_layout_passes=False, use_tc_tiling_on_sc=False),
    )
    def k(idx_hbm, emb_hbm, out_hbm, table_v, idx_v0, idx_v1, rows_v0,
          rows_v1, sem_i0, sem_i1, sem_o0, sem_o1, sem_t):
        idxs_v = (idx_v0, idx_v1)
        rows_v = (rows_v0, rows_v1)
        sem_i = (sem_i0, sem_i1)
        sem_o = (sem_o0, sem_o1)
        wid = lax.axis_index("s") * _NC + lax.axis_index("c")
        base = wid * per_w

        tcp = pltpu.make_async_copy(emb_hbm, table_v, sem_t)
        tcp.start()

        def in_cp(b, buf):
            return pltpu.make_async_copy(
                idx_hbm.at[pl.ds(base + b * blk, blk)], idxs_v[buf],
                sem_i[buf])

        def out_cp(b, buf):
            return pltpu.make_async_copy(
                rows_v[buf],
                out_hbm.at[pl.ds((base + b * blk) * d, blk * d)],
                sem_o[buf])

        in_cp(0, 0).start()
        tcp.wait()

        iota = lax.iota(jnp.int32, _LANES)
        pos0 = iota * d

        def compute(cur):
            @plsc.parallel_loop(0, groups, 1, unroll=8)
            def _(g):
                iv = idxs_v[cur][pl.ds(g * _LANES, _LANES)]
                srcb = iv * d
                gbase = g * (_LANES * d)
                for dd in range(d):
                    vals = plsc.load_gather(table_v, [srcb + dd])
                    plsc.store_scatter(
                        rows_v[cur], [pos0 + (gbase + dd)], vals)

        def block_pair(i, _):
            for j in range(2):
                b = i * 2 + j
                cur = j

                @pl.when(b + 1 < n_blk)
                def _():
                    in_cp(b + 1, 1 - j).start()

                in_cp(b, cur).wait()

                @pl.when(b >= 2)
                def _():
                    out_cp(b - 2, cur).wait()

                compute(cur)
                out_cp(b, cur).start()
            return 0

        lax.fori_loop(0, n_blk // 2, block_pair, 0)
        out_cp(n_blk - 2, (n_blk - 2) % 2).wait()
        out_cp(n_blk - 1, (n_blk - 1) % 2).wait()

    return k


def kernel(idx, emb):
    r, c = idx.shape
    v, d = emb.shape
    n = r * c
    nw = _NC * _NS

    if d == 8 and r % 128 == 0 and c % 8 == 0 and idx.dtype == jnp.int32:
        tpr = r // 128
        n_tiles = c * tpr
        if n_tiles % nw == 0:
            per_w = n_tiles // nw
            tpb = None
            for cand in (128, 64, 32, 16, 8, 4, 2, 1):
                if tpr % cand == 0 and per_w % cand == 0 \
                        and per_w // cand >= 2 \
                        and cand * (128 + d * 128) * 4 * 2 <= 460 * 1024:
                    tpb = cand
                    break
            if tpb is not None:
                idx_n = (idx.transpose(1, 0)
                         .reshape(c // 8, 8, tpr, 128)
                         .transpose(0, 2, 1, 3)
                         .reshape((c // 8) * tpr, 8, 128))
                emb16 = jnp.broadcast_to(
                    emb.reshape(v * d, 1).astype(jnp.float32),
                    (v * d, _LANES)).reshape(v * d * _LANES)
                q = _sc_lookup_native(n_tiles, tpr, v, d, tpb)(idx_n, emb16)
                return (q.reshape(c, tpr, d, 128)
                        .transpose(1, 3, 0, 2)
                        .reshape(r, c, d))

    blk = 16
    for cand in (4096, 2048, 1024, 512, 256, 128, 64, 32, 16):
        per_w = -(-n // nw)
        if n % nw == 0 and per_w % cand == 0 and (per_w // cand) % 2 == 0:
            blk = cand
            break
    chunk = nw * blk * 2
    n_pad = ((n + chunk - 1) // chunk) * chunk

    idx_flat = idx.reshape(n).astype(jnp.int32)
    if n_pad != n:
        idx_flat = jnp.pad(idx_flat, (0, n_pad - n))
    emb_flat = emb.reshape(v * d).astype(jnp.float32)

    out = _sc_lookup_flat(n_pad, v, d, blk)(idx_flat, emb_flat)
    return out[: n * d].reshape(r, c, d)
